# baseline (speedup 1.0000x reference)
import functools

import jax
import jax.numpy as jnp
from jax import lax
from jax.experimental import pallas as pl
from jax.experimental.pallas import tpu as pltpu
from jax.experimental.pallas import tpu_sc as plsc

A = 128
E = 16
C = 256
M = 32
N = 10000
KC = A + E
EPS = 1e-5
SLOPE = 0.01

BN = 200
NB = N // BN
R = BN * M

NC = 2
NS = 16
NW = NC * NS
H = 2
ROWS_H = (N * M) // H
PER_W = ROWS_H // NW
CH = 40
NCH = PER_W // CH
NBH = NB // H
A2 = A // 2


def _leaky(x):
    return jnp.maximum(x, SLOPE * x)


@functools.cache
def _make_sc_gather():
    mesh = plsc.VectorSubcoreMesh(core_axis_name="c", subcore_axis_name="s")

    @functools.partial(
        pl.kernel,
        mesh=mesh,
        out_type=jax.ShapeDtypeStruct((ROWS_H, A), jnp.float32),
        scratch_types=[
            pltpu.VMEM((PER_W,), jnp.int32),
            pltpu.VMEM((2, CH, A), jnp.float32),
            pltpu.VMEM_SHARED((N, A), jnp.float32),
            pltpu.SemaphoreType.DMA,
            pltpu.SemaphoreType.DMA,
            pltpu.SemaphoreType.DMA,
            pltpu.SemaphoreType.DMA,
        ],
    )
    def _sc_gather(atom_hbm, idx_hbm, out_hbm, idx_v, rows_v, table_s,
                   gsem0, gsem1, ssem0, ssem1):
        sid = lax.axis_index("s")
        wid = sid * NC + lax.axis_index("c")
        base = wid * PER_W

        @pl.when(sid == 0)
        def _():
            pltpu.sync_copy(atom_hbm, table_s)

        pltpu.sync_copy(idx_hbm.at[pl.ds(base, PER_W)], idx_v)
        plsc.subcore_barrier()
        gsems = (gsem0, gsem1)
        ssems = (ssem0, ssem1)

        def chunk_idx(i):
            return idx_v.at[pl.ds(i * CH, CH)]

        def out_rows(c):
            return out_hbm.at[pl.ds(base + c * CH, CH)]

        def start_g(c, slot):
            pltpu.async_copy(table_s.at[chunk_idx(c)], rows_v.at[slot],
                             gsems[slot])

        def wait_g(c, slot):
            pltpu.make_async_copy(table_s.at[chunk_idx(c)], rows_v.at[slot],
                                  gsems[slot]).wait()

        def start_st(c, slot):
            pltpu.async_copy(rows_v.at[slot], out_rows(c), ssems[slot])

        def wait_st(c, slot):
            pltpu.make_async_copy(rows_v.at[slot], out_rows(c),
                                  ssems[slot]).wait()

        start_g(0, 0)
        start_g(1, 1)

        def body(j, _):
            wait_g(2 * j, 0)
            start_st(2 * j, 0)
            wait_g(2 * j + 1, 1)
            start_st(2 * j + 1, 1)
            wait_st(2 * j, 0)
            start_g(2 * j + 2, 0)
            wait_st(2 * j + 1, 1)
            start_g(2 * j + 3, 1)
            return 0

        lax.fori_loop(0, (NCH - 3) // 2, body, 0)
        c0, c1, c2 = NCH - 3, NCH - 2, NCH - 1
        wait_g(c0, 0)
        start_st(c0, 0)
        wait_g(c1, 1)
        start_st(c1, 1)
        wait_st(c0, 0)
        start_g(c2, 0)
        wait_st(c1, 1)
        wait_g(c2, 0)
        start_st(c2, 0)
        wait_st(c2, 0)

    return _sc_gather


def _act_block(anbr_ref, nbr_ref, atom_ref, wst_ref, b_ref, wnbr_ref, we_ref):
    s_blk = (
        jnp.dot(atom_ref[...].astype(jnp.bfloat16), wst_ref[...],
                preferred_element_type=jnp.float32)
        + b_ref[...]
    )
    act = (
        jnp.dot(anbr_ref[...].astype(jnp.bfloat16), wnbr_ref[...],
                preferred_element_type=jnp.float32)
        + lax.dot_general(nbr_ref[...], we_ref[...],
                          (((0,), (0,)), ((), ())),
                          preferred_element_type=jnp.float32)
    )
    act = act.reshape(BN, M, C) + s_blk[:, None, :]
    return act.reshape(R, C)


def _stats_body(anbr_ref, nbr_ref, atom_ref, wst_ref, b_ref, wnbr_ref, we_ref, stats_ref):
    i = pl.program_id(0)
    act = _act_block(anbr_ref, nbr_ref, atom_ref, wst_ref, b_ref, wnbr_ref, we_ref)
    s1 = jnp.sum(act, axis=0, keepdims=True)
    s2 = jnp.sum(act * act, axis=0, keepdims=True)
    st = jnp.concatenate([s1, s2], axis=0)

    @pl.when(i == 0)
    def _():
        stats_ref[...] = jnp.zeros_like(stats_ref)

    stats_ref[...] += st


def _gate_body(anbr_ref, nbr_ref, atom_ref, wst_ref, b_ref, wnbr_ref, we_ref,
               stats_ref, g1_ref, b1_ref, ns_ref, st2_ref):
    i = pl.program_id(0)
    cnt = float(N * M)
    mean = stats_ref[0:1, :] / cnt
    var = stats_ref[1:2, :] / cnt - mean * mean
    scale = g1_ref[...] * lax.rsqrt(var + EPS)
    shift = b1_ref[...] - mean * scale
    wnbr_s = (wnbr_ref[...].astype(jnp.float32) * scale).astype(jnp.bfloat16)
    we_s = (we_ref[...].astype(jnp.float32) * scale).astype(jnp.bfloat16)
    s_blk = (
        jnp.dot(atom_ref[...].astype(jnp.bfloat16), wst_ref[...],
                preferred_element_type=jnp.float32)
        + b_ref[...]
    ) * scale + shift
    y = (
        jnp.dot(anbr_ref[...].astype(jnp.bfloat16), wnbr_s,
                preferred_element_type=jnp.float32)
        + lax.dot_general(nbr_ref[...], we_s,
                          (((0,), (0,)), ((), ())),
                          preferred_element_type=jnp.float32)
    )
    y = (y.reshape(BN, M, C) + s_blk[:, None, :]).reshape(R, C)
    f = jax.nn.sigmoid(y[:, :A])
    co = _leaky(y[:, A:])
    ns = (f * co).reshape(BN, M, A).sum(axis=1)
    ns_ref[...] = ns
    s1 = jnp.sum(ns, axis=0, keepdims=True)
    s2 = jnp.sum(ns * ns, axis=0, keepdims=True)
    st = jnp.concatenate([s1, s2], axis=0)

    @pl.when(i == 0)
    def _():
        st2_ref[...] = jnp.zeros_like(st2_ref)

    st2_ref[...] += st


def _final_body(atom_ref, ns_ref, st2_ref, g2_ref, b2_ref, out_ref):
    cnt = float(N)
    mean = st2_ref[0:1, :] / cnt
    var = st2_ref[1:2, :] / cnt - mean * mean
    scale = g2_ref[...] * lax.rsqrt(var + EPS)
    shift = b2_ref[...] - mean * scale
    v = atom_ref[...] + ns_ref[...] * scale + shift
    out_ref[...] = _leaky(v)


def kernel(atom_in_fea, nbr_fea, nbr_fea_idx, W_fc, b_fc,
           bn1_gamma, bn1_beta, bn2_gamma, bn2_beta):
    idx = nbr_fea_idx.astype(jnp.int32).reshape(N * M)
    nbr_t = nbr_fea.reshape(N * M, E).T.astype(jnp.bfloat16)
    wst = W_fc[:, :A].T.astype(jnp.bfloat16)
    wnbr = W_fc[:, A:2 * A].T.astype(jnp.bfloat16)
    we = W_fc[:, 2 * A:].T.astype(jnp.bfloat16)
    b2d = b_fc.reshape(1, C)
    g1 = bn1_gamma.reshape(1, C)
    b1 = bn1_beta.reshape(1, C)
    g2 = bn2_gamma.reshape(1, A)
    b2 = bn2_beta.reshape(1, A)

    gather = _make_sc_gather()
    anbr_h = [gather(atom_in_fea, idx[h * ROWS_H:(h + 1) * ROWS_H])
              for h in range(H)]

    def edge_specs(h):
        return [
            pl.BlockSpec((R, A), lambda i: (i, 0)),
            pl.BlockSpec((E, R), lambda i, h=h: (0, i + h * NBH)),
            pl.BlockSpec((BN, A), lambda i, h=h: (i + h * NBH, 0)),
            pl.BlockSpec((A, C), lambda i: (0, 0)),
            pl.BlockSpec((1, C), lambda i: (0, 0)),
            pl.BlockSpec((A, C), lambda i: (0, 0)),
            pl.BlockSpec((E, C), lambda i: (0, 0)),
        ]

    stats_h = [
        pl.pallas_call(
            _stats_body,
            grid=(NBH,),
            in_specs=edge_specs(h),
            out_specs=pl.BlockSpec((2, C), lambda i: (0, 0)),
            out_shape=jax.ShapeDtypeStruct((2, C), jnp.float32),
        )(anbr_h[h], nbr_t, atom_in_fea, wst, b2d, wnbr, we)
        for h in range(H)
    ]
    stats = sum(stats_h[1:], stats_h[0])

    ns_h, st2_h = [], []
    for h in range(H):
        ns, st2 = pl.pallas_call(
            _gate_body,
            grid=(NBH,),
            in_specs=edge_specs(h) + [
                pl.BlockSpec((2, C), lambda i: (0, 0)),
                pl.BlockSpec((1, C), lambda i: (0, 0)),
                pl.BlockSpec((1, C), lambda i: (0, 0)),
            ],
            out_specs=[
                pl.BlockSpec((BN, A), lambda i: (i, 0)),
                pl.BlockSpec((2, A), lambda i: (0, 0)),
            ],
            out_shape=[
                jax.ShapeDtypeStruct((N // H, A), jnp.float32),
                jax.ShapeDtypeStruct((2, A), jnp.float32),
            ],
        )(anbr_h[h], nbr_t, atom_in_fea, wst, b2d, wnbr, we, stats, g1, b1)
        ns_h.append(ns)
        st2_h.append(st2)
    ns = jnp.concatenate(ns_h, axis=0)
    st2 = sum(st2_h[1:], st2_h[0])

    out = pl.pallas_call(
        _final_body,
        in_specs=[
            pl.BlockSpec((N, A), lambda: (0, 0)),
            pl.BlockSpec((N, A), lambda: (0, 0)),
            pl.BlockSpec((2, A), lambda: (0, 0)),
            pl.BlockSpec((1, A), lambda: (0, 0)),
            pl.BlockSpec((1, A), lambda: (0, 0)),
        ],
        out_specs=pl.BlockSpec((N, A), lambda: (0, 0)),
        out_shape=jax.ShapeDtypeStruct((N, A), jnp.float32),
    )(atom_in_fea, ns, st2, g2, b2)
    return out


# speedup vs baseline: 1.0813x; 1.0813x over previous
import functools

import jax
import jax.numpy as jnp
from jax import lax
from jax.experimental import pallas as pl
from jax.experimental.pallas import tpu as pltpu
from jax.experimental.pallas import tpu_sc as plsc

A = 128
E = 16
C = 256
M = 32
N = 10000
KC = A + E
EPS = 1e-5
SLOPE = 0.01

BN = 200
NB = N // BN
R = BN * M

NC = 2
NS = 16
NW = NC * NS
H = 2
ROWS_H = (N * M) // H
PER_W = ROWS_H // NW
CH = 40
NCH = PER_W // CH
NBH = NB // H
A2 = A // 2


def _leaky(x):
    return jnp.maximum(x, SLOPE * x)


@functools.cache
def _make_sc_gather():
    mesh = plsc.VectorSubcoreMesh(core_axis_name="c", subcore_axis_name="s")

    @functools.partial(
        pl.kernel,
        mesh=mesh,
        out_type=jax.ShapeDtypeStruct((ROWS_H, A), jnp.float32),
        scratch_types=[
            pltpu.VMEM((PER_W,), jnp.int32),
            pltpu.VMEM((2, CH, A), jnp.float32),
            pltpu.VMEM_SHARED((N, A), jnp.float32),
            pltpu.SemaphoreType.DMA,
            pltpu.SemaphoreType.DMA,
            pltpu.SemaphoreType.DMA,
            pltpu.SemaphoreType.DMA,
        ],
    )
    def _sc_gather(atom_hbm, idx_hbm, out_hbm, idx_v, rows_v, table_s,
                   gsem0, gsem1, ssem0, ssem1):
        sid = lax.axis_index("s")
        wid = sid * NC + lax.axis_index("c")
        base = wid * PER_W

        @pl.when(sid == 0)
        def _():
            pltpu.sync_copy(atom_hbm, table_s)

        pltpu.sync_copy(idx_hbm.at[pl.ds(base, PER_W)], idx_v)
        plsc.subcore_barrier()
        gsems = (gsem0, gsem1)
        ssems = (ssem0, ssem1)

        def chunk_idx(i):
            return idx_v.at[pl.ds(i * CH, CH)]

        def out_rows(c):
            return out_hbm.at[pl.ds(base + c * CH, CH)]

        def start_g(c, slot):
            pltpu.async_copy(table_s.at[chunk_idx(c)], rows_v.at[slot],
                             gsems[slot])

        def wait_g(c, slot):
            pltpu.make_async_copy(table_s.at[chunk_idx(c)], rows_v.at[slot],
                                  gsems[slot]).wait()

        def start_st(c, slot):
            pltpu.async_copy(rows_v.at[slot], out_rows(c), ssems[slot])

        def wait_st(c, slot):
            pltpu.make_async_copy(rows_v.at[slot], out_rows(c),
                                  ssems[slot]).wait()

        start_g(0, 0)
        start_g(1, 1)

        def body(j, _):
            wait_g(2 * j, 0)
            start_st(2 * j, 0)
            wait_g(2 * j + 1, 1)
            start_st(2 * j + 1, 1)
            wait_st(2 * j, 0)
            start_g(2 * j + 2, 0)
            wait_st(2 * j + 1, 1)
            start_g(2 * j + 3, 1)
            return 0

        lax.fori_loop(0, (NCH - 3) // 2, body, 0)
        c0, c1, c2 = NCH - 3, NCH - 2, NCH - 1
        wait_g(c0, 0)
        start_st(c0, 0)
        wait_g(c1, 1)
        start_st(c1, 1)
        wait_st(c0, 0)
        start_g(c2, 0)
        wait_st(c1, 1)
        wait_g(c2, 0)
        start_st(c2, 0)
        wait_st(c2, 0)

    return _sc_gather


def _act_block(anbr_ref, nbr_ref, atom_ref, wst_ref, b_ref, wcat_ref):
    s_blk = (
        jnp.dot(atom_ref[...].astype(jnp.bfloat16), wst_ref[...],
                preferred_element_type=jnp.float32)
        + b_ref[...]
    )
    x = jnp.concatenate([anbr_ref[...].astype(jnp.bfloat16), nbr_ref[...]],
                        axis=1)
    act = jnp.dot(x, wcat_ref[...], preferred_element_type=jnp.float32)
    act = act.reshape(BN, M, C) + s_blk[:, None, :]
    return act.reshape(R, C)


def _stats_body(anbr_ref, nbr_ref, atom_ref, wst_ref, b_ref, wcat_ref, stats_ref):
    i = pl.program_id(0)
    act = _act_block(anbr_ref, nbr_ref, atom_ref, wst_ref, b_ref, wcat_ref)
    s1 = jnp.sum(act, axis=0, keepdims=True)
    s2 = jnp.sum(act * act, axis=0, keepdims=True)
    st = jnp.concatenate([s1, s2], axis=0)

    @pl.when(i == 0)
    def _():
        stats_ref[...] = jnp.zeros_like(stats_ref)

    stats_ref[...] += st


def _gate_body(anbr_ref, nbr_ref, atom_ref, wst_ref, b_ref, wcat_ref,
               stats_ref, g1_ref, b1_ref, ns_ref, st2_ref):
    i = pl.program_id(0)
    cnt = float(N * M)
    mean = stats_ref[0:1, :] / cnt
    var = stats_ref[1:2, :] / cnt - mean * mean
    scale = g1_ref[...] * lax.rsqrt(var + EPS)
    shift = b1_ref[...] - mean * scale
    wcat_s = (wcat_ref[...].astype(jnp.float32) * scale).astype(jnp.bfloat16)
    s_blk = (
        jnp.dot(atom_ref[...].astype(jnp.bfloat16), wst_ref[...],
                preferred_element_type=jnp.float32)
        + b_ref[...]
    ) * scale + shift
    x = jnp.concatenate([anbr_ref[...].astype(jnp.bfloat16), nbr_ref[...]],
                        axis=1)
    y = jnp.dot(x, wcat_s, preferred_element_type=jnp.float32)
    y = (y.reshape(BN, M, C) + s_blk[:, None, :]).reshape(R, C)
    f = jax.nn.sigmoid(y[:, :A])
    co = _leaky(y[:, A:])
    ns = (f * co).reshape(BN, M, A).sum(axis=1)
    ns_ref[...] = ns
    s1 = jnp.sum(ns, axis=0, keepdims=True)
    s2 = jnp.sum(ns * ns, axis=0, keepdims=True)
    st = jnp.concatenate([s1, s2], axis=0)

    @pl.when(i == 0)
    def _():
        st2_ref[...] = jnp.zeros_like(st2_ref)

    st2_ref[...] += st


def _final_body(atom_ref, ns_ref, st2_ref, g2_ref, b2_ref, out_ref):
    cnt = float(N)
    mean = st2_ref[0:1, :] / cnt
    var = st2_ref[1:2, :] / cnt - mean * mean
    scale = g2_ref[...] * lax.rsqrt(var + EPS)
    shift = b2_ref[...] - mean * scale
    v = atom_ref[...] + ns_ref[...] * scale + shift
    out_ref[...] = _leaky(v)


def kernel(atom_in_fea, nbr_fea, nbr_fea_idx, W_fc, b_fc,
           bn1_gamma, bn1_beta, bn2_gamma, bn2_beta):
    idx = nbr_fea_idx.astype(jnp.int32).reshape(N * M)
    nbr_flat = nbr_fea.reshape(N * M, E).astype(jnp.bfloat16)
    wst = W_fc[:, :A].T.astype(jnp.bfloat16)
    wcat = jnp.concatenate([W_fc[:, A:2 * A], W_fc[:, 2 * A:]],
                           axis=1).T.astype(jnp.bfloat16)
    b2d = b_fc.reshape(1, C)
    g1 = bn1_gamma.reshape(1, C)
    b1 = bn1_beta.reshape(1, C)
    g2 = bn2_gamma.reshape(1, A)
    b2 = bn2_beta.reshape(1, A)

    gather = _make_sc_gather()
    anbr_h = [gather(atom_in_fea, idx[h * ROWS_H:(h + 1) * ROWS_H])
              for h in range(H)]

    def edge_specs(h):
        return [
            pl.BlockSpec((R, A), lambda i: (i, 0)),
            pl.BlockSpec((R, E), lambda i, h=h: (i + h * NBH, 0)),
            pl.BlockSpec((BN, A), lambda i, h=h: (i + h * NBH, 0)),
            pl.BlockSpec((A, C), lambda i: (0, 0)),
            pl.BlockSpec((1, C), lambda i: (0, 0)),
            pl.BlockSpec((KC, C), lambda i: (0, 0)),
        ]

    stats_h = [
        pl.pallas_call(
            _stats_body,
            grid=(NBH,),
            in_specs=edge_specs(h),
            out_specs=pl.BlockSpec((2, C), lambda i: (0, 0)),
            out_shape=jax.ShapeDtypeStruct((2, C), jnp.float32),
        )(anbr_h[h], nbr_flat, atom_in_fea, wst, b2d, wcat)
        for h in range(H)
    ]
    stats = sum(stats_h[1:], stats_h[0])

    ns_h, st2_h = [], []
    for h in range(H):
        ns, st2 = pl.pallas_call(
            _gate_body,
            grid=(NBH,),
            in_specs=edge_specs(h) + [
                pl.BlockSpec((2, C), lambda i: (0, 0)),
                pl.BlockSpec((1, C), lambda i: (0, 0)),
                pl.BlockSpec((1, C), lambda i: (0, 0)),
            ],
            out_specs=[
                pl.BlockSpec((BN, A), lambda i: (i, 0)),
                pl.BlockSpec((2, A), lambda i: (0, 0)),
            ],
            out_shape=[
                jax.ShapeDtypeStruct((N // H, A), jnp.float32),
                jax.ShapeDtypeStruct((2, A), jnp.float32),
            ],
        )(anbr_h[h], nbr_flat, atom_in_fea, wst, b2d, wcat, stats, g1, b1)
        ns_h.append(ns)
        st2_h.append(st2)
    ns = jnp.concatenate(ns_h, axis=0)
    st2 = sum(st2_h[1:], st2_h[0])

    out = pl.pallas_call(
        _final_body,
        in_specs=[
            pl.BlockSpec((N, A), lambda: (0, 0)),
            pl.BlockSpec((N, A), lambda: (0, 0)),
            pl.BlockSpec((2, A), lambda: (0, 0)),
            pl.BlockSpec((1, A), lambda: (0, 0)),
            pl.BlockSpec((1, A), lambda: (0, 0)),
        ],
        out_specs=pl.BlockSpec((N, A), lambda: (0, 0)),
        out_shape=jax.ShapeDtypeStruct((N, A), jnp.float32),
    )(atom_in_fea, ns, st2, g2, b2)
    return out
